# speedup vs baseline: 4.2025x; 4.2025x over previous
import jax
import jax.numpy as jnp
from jax.experimental import pallas as pl
from jax.experimental.pallas import tpu as pltpu

B, T, C = 32, 720, 512
P_LEN = 24
P_HIST, P_FUT = T // P_LEN, 14
HIDDEN, EMB = 256, 16
N_OUT = 4
SIG_MIN, EPS = 1e-3, 1e-6

ROWS = 2
TILE = ROWS * C
GRID = B // ROWS


def _fused_kernel(x_ref, eWih_ref, eWhh_ref, ebih_ref, ebhh_ref,
                  dWih_ref, dWhh_ref, dbih_ref, dbhh_ref,
                  feT_ref, hW_ref, hb_ref,
                  xnorm_ref, pred_ref):
    x = x_ref[...]

    a1 = 0.5 * (jnp.concatenate([x[:, 1:2], x[:, :T - 1]], axis=1) + x)
    l2 = jnp.concatenate([a1[:, 1:2], a1[:, :T - 1]], axis=1)
    r2 = jnp.concatenate([a1[:, 1:], a1[:, T - 2:T - 1]], axis=1)
    a2 = 0.5 * (l2 + r2)
    l3 = jnp.concatenate([a2[:, 2:3], a2[:, 1:2], a2[:, :T - 2]], axis=1)
    r3 = jnp.concatenate([a2[:, 2:], a2[:, T - 2:T - 1], a2[:, T - 3:T - 2]],
                         axis=1)
    a3 = 0.5 * (l3 + r3)

    hband = x - a1
    d2 = a1 - a2
    d3 = a2 - a3
    low = a3

    def patch(s):
        return s.reshape(ROWS, P_HIST, P_LEN, C)

    def mu_sig_e(s):
        r = patch(s)
        mu = r.mean(axis=2)
        dcen = r - mu[:, :, None, :]
        var = (dcen * dcen).sum(axis=2) * (1.0 / (P_LEN - 1))
        sig = jnp.maximum(jnp.sqrt(var), SIG_MIN)
        e = (r * r).mean(axis=2)
        return mu, sig, e

    mu_L, sig_L, E_L = mu_sig_e(low)
    _, sig_D2, E_D2 = mu_sig_e(d2)
    _, sig_D3, E_D3 = mu_sig_e(d3)
    E_H = (patch(hband) ** 2).mean(axis=2)
    rho_H = E_H / (E_L + E_H + E_D2 + E_D3 + EPS)

    mu_g = low.mean(axis=1)
    dg = low - mu_g[:, None, :]
    sig_g = jnp.maximum(jnp.sqrt((dg * dg).sum(axis=1) * (1.0 / (T - 1))),
                        SIG_MIN)

    log_sig_L = jnp.log(sig_L)
    log_sig_D2 = jnp.log(sig_D2)
    log_sig_D3 = jnp.log(sig_D3)

    xn = ((patch(low) - mu_L[:, :, None, :]) / sig_L[:, :, None, :]
          + patch(hband)
          + patch(d2) / sig_D2[:, :, None, :]
          + patch(d3) / sig_D3[:, :, None, :])
    xnorm_ref[...] = xn.reshape(ROWS, T, C)

    ones_g = jnp.ones((P_HIST, 1), jnp.float32)

    def lanes(g):
        return jnp.concatenate([g[i] for i in range(ROWS)], axis=-1)

    feats = [lanes(mu_L), lanes(log_sig_L), lanes(log_sig_D2),
             lanes(log_sig_D3), lanes(rho_H),
             ones_g * jnp.concatenate([mu_g[i:i + 1] for i in range(ROWS)],
                                      axis=1),
             ones_g * jnp.concatenate(
                 [jnp.log(sig_g)[i:i + 1] for i in range(ROWS)], axis=1)]
    xs = jnp.stack(feats, axis=1)

    last = jnp.concatenate(
        [jnp.concatenate([g[i:i + 1, P_HIST - 1, :] for i in range(ROWS)],
                         axis=1)
         for g in (mu_L, log_sig_L, log_sig_D2, log_sig_D3)], axis=0)

    eWih = eWih_ref[...]
    eWhh = eWhh_ref[...]
    ebih = ebih_ref[...]
    ebhh = ebhh_ref[...]
    H = HIDDEN

    def gru_step(gi, gh, h):
        r = jax.nn.sigmoid(gi[:H] + gh[:H])
        z = jax.nn.sigmoid(gi[H:2 * H] + gh[H:2 * H])
        n = jnp.tanh(gi[2 * H:] + r * gh[2 * H:])
        return (1.0 - z) * n + z * h

    h = jnp.zeros((H, TILE), jnp.float32)
    for t in range(P_HIST):
        gi = jnp.dot(eWih, xs[t], preferred_element_type=jnp.float32) + ebih
        gh = jnp.dot(eWhh, h, preferred_element_type=jnp.float32) + ebhh
        h = gru_step(gi, gh, h)

    dWhh = dWhh_ref[...]
    dbhh = dbhh_ref[...]
    gi_dec = (jnp.dot(dWih_ref[...], feT_ref[...],
                      preferred_element_type=jnp.float32) + dbih_ref[...])
    hW = hW_ref[...]
    hb = hb_ref[...]

    outs = []
    for t in range(P_FUT):
        gh = jnp.dot(dWhh, h, preferred_element_type=jnp.float32) + dbhh
        h = gru_step(gi_dec[:, t:t + 1], gh, h)
        outs.append(jnp.dot(hW, h, preferred_element_type=jnp.float32)
                    + hb + last)
    pred_ref[...] = jnp.concatenate(outs, axis=0)


def kernel(x, enc_Wih, enc_Whh, enc_bih, enc_bhh,
           dec_Wih, dec_Whh, dec_bih, dec_bhh,
           future_embed, head_W, head_b):
    full = lambda shape: pl.BlockSpec(shape, lambda i: tuple(0 for _ in shape))
    x_norm, pred_raw = pl.pallas_call(
        _fused_kernel,
        grid=(GRID,),
        in_specs=[
            pl.BlockSpec((ROWS, T, C), lambda i: (i, 0, 0)),
            full((3 * HIDDEN, 7)),
            full((3 * HIDDEN, HIDDEN)),
            full((3 * HIDDEN, 1)),
            full((3 * HIDDEN, 1)),
            full((3 * HIDDEN, EMB)),
            full((3 * HIDDEN, HIDDEN)),
            full((3 * HIDDEN, 1)),
            full((3 * HIDDEN, 1)),
            full((EMB, P_FUT)),
            full((N_OUT, HIDDEN)),
            full((N_OUT, 1)),
        ],
        out_specs=[
            pl.BlockSpec((ROWS, T, C), lambda i: (i, 0, 0)),
            pl.BlockSpec((P_FUT * N_OUT, TILE), lambda i: (0, i)),
        ],
        out_shape=[
            jax.ShapeDtypeStruct((B, T, C), jnp.float32),
            jax.ShapeDtypeStruct((P_FUT * N_OUT, B * C), jnp.float32),
        ],
        compiler_params=pltpu.CompilerParams(
            dimension_semantics=("parallel",),
        ),
        name="wave_band_norm_fused",
    )(x, enc_Wih, enc_Whh, enc_bih.reshape(-1, 1), enc_bhh.reshape(-1, 1),
      dec_Wih, dec_Whh, dec_bih.reshape(-1, 1), dec_bhh.reshape(-1, 1),
      future_embed.T, head_W, head_b.reshape(-1, 1))

    pred = pred_raw.reshape(P_FUT, N_OUT, B, C).transpose(2, 0, 1, 3)
    return x_norm, pred


# speedup vs baseline: 4.2958x; 1.0222x over previous
import jax
import jax.numpy as jnp
from jax.experimental import pallas as pl
from jax.experimental.pallas import tpu as pltpu

B, T, C = 32, 720, 512
P_LEN = 24
P_HIST, P_FUT = T // P_LEN, 14
HIDDEN, EMB = 256, 16
N_OUT = 4
SIG_MIN, EPS = 1e-3, 1e-6

ROWS = 2
TILE = ROWS * C
GRID = B // ROWS


def _fused_kernel(x_ref, eWih_ref, eWhh_ref, ebih_ref, ebhh_ref,
                  dWih_ref, dWhh_ref, dbih_ref, dbhh_ref,
                  feT_ref, hW_ref, hb_ref,
                  xnorm_ref, pred_ref):
    x = x_ref[...]

    a1 = 0.5 * (jnp.concatenate([x[:, 1:2], x[:, :T - 1]], axis=1) + x)
    l2 = jnp.concatenate([a1[:, 1:2], a1[:, :T - 1]], axis=1)
    r2 = jnp.concatenate([a1[:, 1:], a1[:, T - 2:T - 1]], axis=1)
    a2 = 0.5 * (l2 + r2)
    l3 = jnp.concatenate([a2[:, 2:3], a2[:, 1:2], a2[:, :T - 2]], axis=1)
    r3 = jnp.concatenate([a2[:, 2:], a2[:, T - 2:T - 1], a2[:, T - 3:T - 2]],
                         axis=1)
    a3 = 0.5 * (l3 + r3)

    hband = x - a1
    d2 = a1 - a2
    d3 = a2 - a3
    low = a3

    def patch(s):
        return s.reshape(ROWS, P_HIST, P_LEN, C)

    def mu_sig_e(s):
        r = patch(s)
        mu = r.mean(axis=2)
        dcen = r - mu[:, :, None, :]
        var = (dcen * dcen).sum(axis=2) * (1.0 / (P_LEN - 1))
        sig = jnp.maximum(jnp.sqrt(var), SIG_MIN)
        e = (r * r).mean(axis=2)
        return mu, sig, e

    mu_L, sig_L, E_L = mu_sig_e(low)
    _, sig_D2, E_D2 = mu_sig_e(d2)
    _, sig_D3, E_D3 = mu_sig_e(d3)
    E_H = (patch(hband) ** 2).mean(axis=2)
    rho_H = E_H / (E_L + E_H + E_D2 + E_D3 + EPS)

    mu_g = low.mean(axis=1)
    dg = low - mu_g[:, None, :]
    sig_g = jnp.maximum(jnp.sqrt((dg * dg).sum(axis=1) * (1.0 / (T - 1))),
                        SIG_MIN)

    log_sig_L = jnp.log(sig_L)
    log_sig_D2 = jnp.log(sig_D2)
    log_sig_D3 = jnp.log(sig_D3)

    xn = ((patch(low) - mu_L[:, :, None, :]) / sig_L[:, :, None, :]
          + patch(hband)
          + patch(d2) / sig_D2[:, :, None, :]
          + patch(d3) / sig_D3[:, :, None, :])
    xnorm_ref[...] = xn.reshape(ROWS, T, C)

    ones_g = jnp.ones((P_HIST, 1), jnp.float32)

    def lanes(g):
        return jnp.concatenate([g[i] for i in range(ROWS)], axis=-1)

    feats = [lanes(mu_L), lanes(log_sig_L), lanes(log_sig_D2),
             lanes(log_sig_D3), lanes(rho_H),
             ones_g * jnp.concatenate([mu_g[i:i + 1] for i in range(ROWS)],
                                      axis=1),
             ones_g * jnp.concatenate(
                 [jnp.log(sig_g)[i:i + 1] for i in range(ROWS)], axis=1)]
    xs = jnp.stack(feats, axis=1)

    last = jnp.concatenate(
        [jnp.concatenate([g[i:i + 1, P_HIST - 1, :] for i in range(ROWS)],
                         axis=1)
         for g in (mu_L, log_sig_L, log_sig_D2, log_sig_D3)], axis=0)

    bf = jnp.bfloat16
    eWih = eWih_ref[...].astype(bf)
    eWhh = eWhh_ref[...].astype(bf)
    ebih = ebih_ref[...]
    ebhh = ebhh_ref[...]
    H = HIDDEN

    def gru_step(gi, gh, h):
        r = jax.nn.sigmoid(gi[:H] + gh[:H])
        z = jax.nn.sigmoid(gi[H:2 * H] + gh[H:2 * H])
        n = jnp.tanh(gi[2 * H:] + r * gh[2 * H:])
        return (1.0 - z) * n + z * h

    xs = xs.astype(bf)
    h = jnp.zeros((H, TILE), jnp.float32)
    for t in range(P_HIST):
        gi = jnp.dot(eWih, xs[t], preferred_element_type=jnp.float32) + ebih
        gh = jnp.dot(eWhh, h.astype(bf),
                     preferred_element_type=jnp.float32) + ebhh
        h = gru_step(gi, gh, h)

    dWhh = dWhh_ref[...].astype(bf)
    dbhh = dbhh_ref[...]
    gi_dec = (jnp.dot(dWih_ref[...], feT_ref[...],
                      preferred_element_type=jnp.float32) + dbih_ref[...])
    hW = hW_ref[...]
    hb = hb_ref[...]

    outs = []
    for t in range(P_FUT):
        gh = jnp.dot(dWhh, h.astype(bf),
                     preferred_element_type=jnp.float32) + dbhh
        h = gru_step(gi_dec[:, t:t + 1], gh, h)
        outs.append(jnp.dot(hW, h, preferred_element_type=jnp.float32)
                    + hb + last)
    pred_ref[...] = jnp.concatenate(outs, axis=0)


def kernel(x, enc_Wih, enc_Whh, enc_bih, enc_bhh,
           dec_Wih, dec_Whh, dec_bih, dec_bhh,
           future_embed, head_W, head_b):
    full = lambda shape: pl.BlockSpec(shape, lambda i: tuple(0 for _ in shape))
    x_norm, pred_raw = pl.pallas_call(
        _fused_kernel,
        grid=(GRID,),
        in_specs=[
            pl.BlockSpec((ROWS, T, C), lambda i: (i, 0, 0)),
            full((3 * HIDDEN, 7)),
            full((3 * HIDDEN, HIDDEN)),
            full((3 * HIDDEN, 1)),
            full((3 * HIDDEN, 1)),
            full((3 * HIDDEN, EMB)),
            full((3 * HIDDEN, HIDDEN)),
            full((3 * HIDDEN, 1)),
            full((3 * HIDDEN, 1)),
            full((EMB, P_FUT)),
            full((N_OUT, HIDDEN)),
            full((N_OUT, 1)),
        ],
        out_specs=[
            pl.BlockSpec((ROWS, T, C), lambda i: (i, 0, 0)),
            pl.BlockSpec((P_FUT * N_OUT, TILE), lambda i: (0, i)),
        ],
        out_shape=[
            jax.ShapeDtypeStruct((B, T, C), jnp.float32),
            jax.ShapeDtypeStruct((P_FUT * N_OUT, B * C), jnp.float32),
        ],
        compiler_params=pltpu.CompilerParams(
            dimension_semantics=("parallel",),
        ),
        name="wave_band_norm_fused",
    )(x, enc_Wih, enc_Whh, enc_bih.reshape(-1, 1), enc_bhh.reshape(-1, 1),
      dec_Wih, dec_Whh, dec_bih.reshape(-1, 1), dec_bhh.reshape(-1, 1),
      future_embed.T, head_W, head_b.reshape(-1, 1))

    pred = pred_raw.reshape(P_FUT, N_OUT, B, C).transpose(2, 0, 1, 3)
    return x_norm, pred
